# baseline (speedup 1.0000x reference)
import functools

import jax
import jax.numpy as jnp
from jax import lax
from jax.experimental import pallas as pl
from jax.experimental.pallas import tpu as pltpu
from jax.experimental.pallas import tpu_sc as plsc

VOCAB = 32768
HIDDEN = 256
B, L = 64, 32
NTOK = B * L

VB = 2048


def _gather_sc(embed_table, flat_ids):
    n = flat_ids.shape[0]
    info = plsc.get_sparse_core_info()
    ncores = 1
    nw = ncores * info.num_subcores
    b_per_w = n // nw
    mesh = plsc.VectorSubcoreMesh(
        core_axis_name="c", subcore_axis_name="s", num_cores=ncores
    )

    @functools.partial(
        pl.kernel,
        out_type=jax.ShapeDtypeStruct((n, HIDDEN), jnp.float32),
        mesh=mesh,
        scratch_types=[
            pltpu.VMEM((b_per_w,), jnp.int32),
            pltpu.VMEM((b_per_w, HIDDEN), jnp.float32),
            pltpu.SemaphoreType.DMA,
        ],
    )
    def gather_kernel(table_hbm, idx_hbm, out_hbm, idx_v, rows_v, sem):
        wid = lax.axis_index("s") * ncores + lax.axis_index("c")
        base = wid * b_per_w
        pltpu.sync_copy(idx_hbm.at[pl.ds(base, b_per_w)], idx_v)
        pltpu.async_copy(table_hbm.at[idx_v], rows_v, sem).wait()
        pltpu.sync_copy(rows_v, out_hbm.at[pl.ds(base, b_per_w)])

    return gather_kernel(embed_table, flat_ids)


HALF = NTOK // 2


def _mm_kernel(h0_ref, h1_ref, w_ref, out_ref):
    wb = w_ref[...].astype(jnp.bfloat16)
    out_ref[:HALF, :] = lax.dot_general(
        h0_ref[...].astype(jnp.bfloat16), wb, (((1,), (1,)), ((), ())),
        preferred_element_type=jnp.float32,
    )
    out_ref[HALF:, :] = lax.dot_general(
        h1_ref[...].astype(jnp.bfloat16), wb, (((1,), (1,)), ((), ())),
        preferred_element_type=jnp.float32,
    )


def _matmul_tc(h0, h1, head_w):
    return pl.pallas_call(
        _mm_kernel,
        grid=(VOCAB // VB,),
        in_specs=[
            pl.BlockSpec((HALF, HIDDEN), lambda i: (0, 0)),
            pl.BlockSpec((HALF, HIDDEN), lambda i: (0, 0)),
            pl.BlockSpec((VB, HIDDEN), lambda i: (i, 0)),
        ],
        out_specs=pl.BlockSpec((NTOK, VB), lambda i: (0, i)),
        out_shape=jax.ShapeDtypeStruct((NTOK, VOCAB), jnp.float32),
    )(h0, h1, head_w)


def kernel(input_ids, embed_table, head_w):
    flat_ids = input_ids.reshape(NTOK).astype(jnp.int32)
    h0 = _gather_sc(embed_table, flat_ids[:HALF])
    h1 = _gather_sc(embed_table, flat_ids[HALF:])
    logits = _matmul_tc(h0, h1, head_w)
    return logits.reshape(B, L, VOCAB)


# speedup vs baseline: 1.0513x; 1.0513x over previous
import functools

import jax
import jax.numpy as jnp
from jax import lax
from jax.experimental import pallas as pl
from jax.experimental.pallas import tpu as pltpu
from jax.experimental.pallas import tpu_sc as plsc

VOCAB = 32768
HIDDEN = 256
B, L = 64, 32
NTOK = B * L

VB = 2048


def _gather_sc(embed_table, flat_ids):
    n = flat_ids.shape[0]
    info = plsc.get_sparse_core_info()
    ncores = 1
    nw = ncores * info.num_subcores
    b_per_w = n // nw
    mesh = plsc.VectorSubcoreMesh(
        core_axis_name="c", subcore_axis_name="s", num_cores=ncores
    )

    half = b_per_w // 2

    @functools.partial(
        pl.kernel,
        out_type=jax.ShapeDtypeStruct((n, HIDDEN), jnp.float32),
        mesh=mesh,
        scratch_types=[
            pltpu.VMEM((half,), jnp.int32),
            pltpu.VMEM((half,), jnp.int32),
            pltpu.VMEM((half, HIDDEN), jnp.float32),
            pltpu.VMEM((half, HIDDEN), jnp.float32),
            pltpu.SemaphoreType.DMA,
            pltpu.SemaphoreType.DMA,
            pltpu.SemaphoreType.DMA,
            pltpu.SemaphoreType.DMA,
        ],
    )
    def gather_kernel(table_hbm, idx_hbm, out_hbm, idx0, idx1, rows0, rows1,
                      s0, s1, s2, s3):
        wid = lax.axis_index("s") * ncores + lax.axis_index("c")
        base = wid * b_per_w
        pltpu.sync_copy(idx_hbm.at[pl.ds(base, half)], idx0)
        g0 = pltpu.async_copy(table_hbm.at[idx0], rows0, s0)
        pltpu.sync_copy(idx_hbm.at[pl.ds(base + half, half)], idx1)
        g1 = pltpu.async_copy(table_hbm.at[idx1], rows1, s1)
        g0.wait()
        w0 = pltpu.async_copy(rows0, out_hbm.at[pl.ds(base, half)], s2)
        g1.wait()
        w1 = pltpu.async_copy(rows1, out_hbm.at[pl.ds(base + half, half)], s3)
        w0.wait()
        w1.wait()

    return gather_kernel(embed_table, flat_ids)


def _mm_kernel(h_ref, w_ref, out_ref):
    hb = h_ref[...].astype(jnp.bfloat16)
    wb = w_ref[...].astype(jnp.bfloat16)
    out_ref[...] = lax.dot_general(
        hb, wb, (((1,), (1,)), ((), ())), preferred_element_type=jnp.float32
    )


def _matmul_tc(h, head_w):
    return pl.pallas_call(
        _mm_kernel,
        grid=(VOCAB // VB,),
        in_specs=[
            pl.BlockSpec((NTOK, HIDDEN), lambda i: (0, 0)),
            pl.BlockSpec((VB, HIDDEN), lambda i: (i, 0)),
        ],
        out_specs=pl.BlockSpec((NTOK, VB), lambda i: (0, i)),
        out_shape=jax.ShapeDtypeStruct((NTOK, VOCAB), jnp.float32),
    )(h, head_w)


def kernel(input_ids, embed_table, head_w):
    flat_ids = input_ids.reshape(NTOK).astype(jnp.int32)
    h = _gather_sc(embed_table, flat_ids)
    logits = _matmul_tc(h, head_w)
    return logits.reshape(B, L, VOCAB)
